# Initial kernel scaffold; baseline (speedup 1.0000x reference)
import jax
import jax.numpy as jnp
from jax.experimental import pallas as pl


def kernel(x, params, edge_index, batch):
    raise NotImplementedError("write your pallas kernel here")


# speedup vs baseline: 2.1326x; 2.1326x over previous
import jax
import jax.numpy as jnp
from jax import lax
from jax.experimental import pallas as pl
from jax.experimental.pallas import tpu as pltpu

B = 50
NPG = 1000
KP = 500
PN = 512
KNN = 4
NB = 50176
N_NODES = 50000
N_EDGES = 1600000

_BF = jnp.bfloat16
_F32 = jnp.float32


def _lrelu(v):
    return jnp.where(v >= 0, v, v * jnp.float32(0.01))


def _bf(x):
    return x.astype(_BF)


def _split3(x):
    h = x.astype(_BF)
    r = x - h.astype(_F32)
    m = r.astype(_BF)
    l = (r - m.astype(_F32)).astype(_BF)
    return h, m, l


def _dot(a, b):
    return jax.lax.dot(a, b, preferred_element_type=_F32)


def _mm(a, b):
    return _dot(_bf(a), _bf(b))


def _mm_onehot(o, b):
    obf = o.astype(_BF)
    bh, bm, bl = _split3(b)
    return _dot(obf, bh) + _dot(obf, bm) + _dot(obf, bl)


def _tr(x, eye_bf):
    xh, xm, xl = _split3(x)
    dn = (((0,), (0,)), ((), ()))
    f = lambda v: lax.dot_general(v, eye_bf, dn, preferred_element_type=_F32)
    return f(xh) + f(xm) + f(xl)


def _key_i32(score):
    u = lax.bitcast_convert_type(score, jnp.int32)
    return jnp.where(u >= 0, u, jnp.int32(-2147483648) - u)


def _conv(h, d2_base, lane_iota, w1at, w1bt, w2t, b1, b2):
    hh, hm, hl = _split3(h)
    xi_term = _dot(hh, _bf(w1at))
    b1r = b1[...]
    b2r = b2[...]
    w1bt_bf = _bf(w1bt)
    w2t_bf = _bf(w2t)
    out = jnp.zeros((PN, w2t.shape[1]), _F32)
    d2 = d2_base
    for _ in range(KNN):
        mv = jnp.min(d2, axis=1, keepdims=True)
        am = jnp.min(jnp.where(d2 == mv, lane_iota, jnp.float32(9e9)),
                     axis=1, keepdims=True)
        sel = lane_iota == am
        d2 = jnp.where(sel, jnp.float32(1e10), d2)
        ob = jnp.where(sel, 1.0, 0.0).astype(_BF)
        xj = (_dot(ob, hh) + _dot(ob, hm) + _dot(ob, hl))
        diff = xj - h
        h1 = _lrelu(xi_term + _dot(_bf(diff), w1bt_bf) + b1r)
        h2 = _lrelu(_dot(_bf(h1), w2t_bf) + b2r)
        out = out + h2
    return out


def _d2_of(h, eye_bf, pad_big):
    p3t = _tr(h[:, 0:3], eye_bf)
    d0 = h[:, 0:1] - p3t[0:1, :]
    d1 = h[:, 1:2] - p3t[1:2, :]
    d2c = h[:, 2:3] - p3t[2:3, :]
    d2 = (d0 * d0 + d1 * d1) + d2c * d2c
    return d2 + pad_big


def _graph_body(xg_ref, xgt_ref, agg_ref,
                wrel_ref, wroot_ref, relb_ref,
                c1a_ref, c1b_ref, c1w2_ref, c1b1_ref, c1b2_ref,
                c2a_ref, c2b_ref, c2w2_ref, c2b1_ref, c2b2_ref,
                c3a_ref, c3b_ref, c3w2_ref, c3b1_ref, c3b2_ref,
                c4a_ref, c4b_ref, c4w2_ref, c4b1_ref, c4b2_ref,
                n1x_ref, n1a_ref, n1b_ref, n1c_ref, n1d_ref, n1bias_ref,
                n2_ref, n2b_ref,
                n3a_ref, n3b_ref, n3c_ref, n3d_ref, n3bias_ref,
                n4_ref, n4b_ref,
                out_ref):
    xg = xg_ref[0]
    xgt = xgt_ref[0]

    acc = jnp.zeros((1, NPG), _F32)
    for d in range(4):
        agg_d = agg_ref[0, d:d + 1, :] + agg_ref[0, 4 + d:5 + d, :]
        acc = acc + agg_d * wrel_ref[0, d]
    root = jnp.zeros((1, NPG), _F32)
    for d in range(4):
        root = root + xgt[d:d + 1, :] * wroot_ref[0, d]
    score = _lrelu(acc + relb_ref[0, 0] + root)

    keys = _key_i32(score)
    lo0 = jnp.min(keys)
    hi0 = jnp.max(keys) + 1

    def bis(_, lh):
        lo, hi = lh
        mid = (lo >> 1) + (hi >> 1) + (lo & hi & 1)
        c = jnp.sum(jnp.where(keys >= mid, 1, 0).astype(jnp.int32))
        big = c >= KP
        return jnp.where(big, mid, lo), jnp.where(big, hi, mid)

    v500, _ = lax.fori_loop(0, 34, bis, (lo0, hi0))
    gt = keys > v500
    eq = keys == v500
    cnt_gt = jnp.sum(jnp.where(gt, 1, 0).astype(jnp.int32))
    quota = (KP - cnt_gt).astype(_F32)

    tri = (lax.broadcasted_iota(jnp.int32, (NPG, NPG), 0)
           <= lax.broadcasted_iota(jnp.int32, (NPG, NPG), 1)).astype(_BF)
    csum_eq = _dot(jnp.where(eq, 1.0, 0.0).astype(_BF), tri)
    mask = gt | (eq & (csum_eq <= quota))
    maskf = jnp.where(mask, 1.0, 0.0)
    pos = _dot(maskf.astype(_BF), tri) - 1.0

    sub_iota = lax.broadcasted_iota(jnp.int32, (PN, NPG), 0).astype(_F32)
    cf = jnp.where(mask & (pos == sub_iota), 1.0, 0.0)
    x0r = _mm_onehot(cf, xg)
    ones11 = jnp.ones((1, 1), _BF)
    dn = (((0,), (0,)), ((), ()))
    sh, sm_, sl = _split3(score)
    score_col = (lax.dot_general(sh, ones11, dn, preferred_element_type=_F32)
                 + lax.dot_general(sm_, ones11, dn, preferred_element_type=_F32)
                 + lax.dot_general(sl, ones11, dn, preferred_element_type=_F32))
    s0 = _mm_onehot(cf, score_col)
    x0 = x0r * s0

    eye_bf = (lax.broadcasted_iota(jnp.int32, (PN, PN), 0)
              == lax.broadcasted_iota(jnp.int32, (PN, PN), 1)).astype(_BF)
    lane_iota = lax.broadcasted_iota(jnp.int32, (PN, PN), 1).astype(_F32)
    rowi = lax.broadcasted_iota(jnp.int32, (PN, PN), 0)
    coli = lax.broadcasted_iota(jnp.int32, (PN, PN), 1)
    pad_big = jnp.where((rowi == coli) | (coli >= KP),
                        jnp.float32(1e10), jnp.float32(0.0))

    a = _conv(x0, _d2_of(x0, eye_bf, pad_big), lane_iota,
              c1a_ref[...], c1b_ref[...], c1w2_ref[...], c1b1_ref, c1b2_ref)
    bb = _conv(a, _d2_of(a, eye_bf, pad_big), lane_iota,
               c2a_ref[...], c2b_ref[...], c2w2_ref[...], c2b1_ref, c2b2_ref)
    cc = _conv(bb, _d2_of(bb, eye_bf, pad_big), lane_iota,
               c3a_ref[...], c3b_ref[...], c3w2_ref[...], c3b1_ref, c3b2_ref)
    dd = _conv(cc, _d2_of(cc, eye_bf, pad_big), lane_iota,
               c4a_ref[...], c4b_ref[...], c4w2_ref[...], c4b1_ref, c4b2_ref)

    h1 = _lrelu(_mm(x0, n1x_ref[...]) + _mm(a, n1a_ref[...])
                + _mm(bb, n1b_ref[...]) + _mm(cc, n1c_ref[...])
                + _mm(dd, n1d_ref[...]) + n1bias_ref[...])
    h2 = _mm(h1, n2_ref[...]) + n2b_ref[...]
    live = lax.broadcasted_iota(jnp.int32, (PN, 1), 0) < KP
    mx = jnp.max(jnp.where(live, h2, jnp.float32(-1e30)), axis=0, keepdims=True)
    mn = jnp.min(jnp.where(live, h2, jnp.float32(1e30)), axis=0, keepdims=True)
    sm = jnp.sum(jnp.where(live, h2, jnp.float32(0.0)), axis=0, keepdims=True)
    mean = sm / jnp.float32(KP)
    h3 = _lrelu(_mm(_lrelu(mx), n3a_ref[...]) + _mm(_lrelu(mn), n3b_ref[...])
                + _mm(_lrelu(sm), n3c_ref[...]) + _mm(_lrelu(mean), n3d_ref[...])
                + n3bias_ref[...])
    res = jnp.sum(h3 * n4_ref[...], axis=1, keepdims=True) + n4b_ref[...]
    out_ref[0] = jnp.broadcast_to(res, (1, 128))


def _run_graphs(xg3, xgt3, agg3, wrel, wroot, relb, wl):
    full = lambda shape: pl.BlockSpec(shape, lambda g: tuple(0 for _ in shape))
    in_specs = [
        pl.BlockSpec((1, NPG, 4), lambda g: (g, 0, 0)),
        pl.BlockSpec((1, 4, NPG), lambda g: (g, 0, 0)),
        pl.BlockSpec((1, 8, NPG), lambda g: (g, 0, 0)),
        pl.BlockSpec(memory_space=pltpu.SMEM),
        pl.BlockSpec(memory_space=pltpu.SMEM),
        pl.BlockSpec(memory_space=pltpu.SMEM),
    ] + [full(w.shape) for w in wl]
    out = pl.pallas_call(
        _graph_body,
        grid=(B,),
        out_shape=jax.ShapeDtypeStruct((B, 1, 128), _F32),
        in_specs=in_specs,
        out_specs=pl.BlockSpec((1, 1, 128), lambda g: (g, 0, 0)),
    )(xg3, xgt3, agg3, wrel, wroot, relb, *wl)
    return out[:, 0, 0:1]


def _edge_aggregate(x, src, dst):
    a4 = jax.ops.segment_sum(x[src], dst, num_segments=N_NODES)
    return jnp.stack([a4.T, jnp.zeros_like(a4.T)], axis=0)


def kernel(x, params, edge_index, batch):
    p = params
    aggP = _edge_aggregate(x, edge_index[0], edge_index[1])
    agg3 = (aggP.reshape(2, 4, B, NPG).transpose(2, 0, 1, 3)
            .reshape(B, 8, NPG))

    xg3 = x.reshape(B, NPG, 4)
    xgt3 = xg3.swapaxes(1, 2)

    def c_w(i):
        w1 = p['c%d_w1' % i]
        d = w1.shape[1] // 2
        w1a, w1b = w1[:, :d], w1[:, d:]
        return [w1a.T, w1b.T, p['c%d_w2' % i].T,
                p['c%d_b1' % i][None, :], p['c%d_b2' % i][None, :]]

    n1 = p['nn1_w'].T
    n1parts = [n1[0:4], n1[4:196], n1[196:388], n1[388:580], n1[580:772]]
    n3 = p['nn3_w'].T
    n3parts = [n3[0:192], n3[192:384], n3[384:576], n3[576:768]]

    wl = (c_w(1) + c_w(2) + c_w(3) + c_w(4)
          + n1parts + [p['nn1_b'][None, :], p['nn2_w'].T, p['nn2_b'][None, :]]
          + n3parts + [p['nn3_b'][None, :], p['nn4_w'], p['nn4_b'][None, :]])
    wl = [w.astype(_F32) for w in wl]

    return _run_graphs(xg3, xgt3, agg3,
                       p['sag_rel_w'], p['sag_root_w'],
                       p['sag_rel_b'][None, :], wl)


# speedup vs baseline: 9.7186x; 4.5571x over previous
import dataclasses
import functools

import jax
import jax.numpy as jnp
from jax import lax
from jax.experimental import pallas as pl
from jax.experimental.pallas import tpu as pltpu
from jax.experimental.pallas import tpu_sc as plsc

B = 50
NPG = 1000
KP = 500
PN = 512
KNN = 4
NB = 50176
N_NODES = 50000
N_EDGES = 1600000

_BF = jnp.bfloat16
_F32 = jnp.float32


def _lrelu(v):
    return jnp.where(v >= 0, v, v * jnp.float32(0.01))


def _bf(x):
    return x.astype(_BF)


def _split3(x):
    h = x.astype(_BF)
    r = x - h.astype(_F32)
    m = r.astype(_BF)
    l = (r - m.astype(_F32)).astype(_BF)
    return h, m, l


def _dot(a, b):
    return jax.lax.dot(a, b, preferred_element_type=_F32)


def _mm(a, b):
    return _dot(_bf(a), _bf(b))


def _mm_onehot(o, b):
    obf = o.astype(_BF)
    bh, bm, bl = _split3(b)
    return _dot(obf, bh) + _dot(obf, bm) + _dot(obf, bl)


def _tr(x, eye_bf):
    xh, xm, xl = _split3(x)
    dn = (((0,), (0,)), ((), ()))
    f = lambda v: lax.dot_general(v, eye_bf, dn, preferred_element_type=_F32)
    return f(xh) + f(xm) + f(xl)


def _key_i32(score):
    u = lax.bitcast_convert_type(score, jnp.int32)
    return jnp.where(u >= 0, u, jnp.int32(-2147483648) - u)


def _conv(h, d2_base, lane_iota, w1at, w1bt, w2t, b1, b2):
    hh, hm, hl = _split3(h)
    xi_term = _dot(hh, _bf(w1at))
    b1r = b1[...]
    b2r = b2[...]
    w1bt_bf = _bf(w1bt)
    w2t_bf = _bf(w2t)
    out = jnp.zeros((PN, w2t.shape[1]), _F32)
    d2 = d2_base
    for _ in range(KNN):
        mv = jnp.min(d2, axis=1, keepdims=True)
        am = jnp.min(jnp.where(d2 == mv, lane_iota, jnp.float32(9e9)),
                     axis=1, keepdims=True)
        sel = lane_iota == am
        d2 = jnp.where(sel, jnp.float32(1e10), d2)
        ob = jnp.where(sel, 1.0, 0.0).astype(_BF)
        xj = (_dot(ob, hh) + _dot(ob, hm) + _dot(ob, hl))
        diff = xj - h
        h1 = _lrelu(xi_term + _dot(_bf(diff), w1bt_bf) + b1r)
        h2 = _lrelu(_dot(_bf(h1), w2t_bf) + b2r)
        out = out + h2
    return out


def _d2_of(h, eye_bf, pad_big):
    p3t = _tr(h[:, 0:3], eye_bf)
    d0 = h[:, 0:1] - p3t[0:1, :]
    d1 = h[:, 1:2] - p3t[1:2, :]
    d2c = h[:, 2:3] - p3t[2:3, :]
    d2 = (d0 * d0 + d1 * d1) + d2c * d2c
    return d2 + pad_big


def _graph_body(xg_ref, xgt_ref, agg_ref,
                wrel_ref, wroot_ref, relb_ref,
                c1a_ref, c1b_ref, c1w2_ref, c1b1_ref, c1b2_ref,
                c2a_ref, c2b_ref, c2w2_ref, c2b1_ref, c2b2_ref,
                c3a_ref, c3b_ref, c3w2_ref, c3b1_ref, c3b2_ref,
                c4a_ref, c4b_ref, c4w2_ref, c4b1_ref, c4b2_ref,
                n1x_ref, n1a_ref, n1b_ref, n1c_ref, n1d_ref, n1bias_ref,
                n2_ref, n2b_ref,
                n3a_ref, n3b_ref, n3c_ref, n3d_ref, n3bias_ref,
                n4_ref, n4b_ref,
                out_ref):
    xg = xg_ref[0]
    xgt = xgt_ref[0]

    acc = jnp.zeros((1, NPG), _F32)
    for d in range(4):
        agg_d = agg_ref[0, d:d + 1, :] + agg_ref[0, 4 + d:5 + d, :]
        acc = acc + agg_d * wrel_ref[0, d]
    root = jnp.zeros((1, NPG), _F32)
    for d in range(4):
        root = root + xgt[d:d + 1, :] * wroot_ref[0, d]
    score = _lrelu(acc + relb_ref[0, 0] + root)

    keys = _key_i32(score)
    lo0 = jnp.min(keys)
    hi0 = jnp.max(keys) + 1

    def bis(_, lh):
        lo, hi = lh
        mid = (lo >> 1) + (hi >> 1) + (lo & hi & 1)
        c = jnp.sum(jnp.where(keys >= mid, 1, 0).astype(jnp.int32))
        big = c >= KP
        return jnp.where(big, mid, lo), jnp.where(big, hi, mid)

    v500, _ = lax.fori_loop(0, 34, bis, (lo0, hi0))
    gt = keys > v500
    eq = keys == v500
    cnt_gt = jnp.sum(jnp.where(gt, 1, 0).astype(jnp.int32))
    quota = (KP - cnt_gt).astype(_F32)

    tri = (lax.broadcasted_iota(jnp.int32, (NPG, NPG), 0)
           <= lax.broadcasted_iota(jnp.int32, (NPG, NPG), 1)).astype(_BF)
    csum_eq = _dot(jnp.where(eq, 1.0, 0.0).astype(_BF), tri)
    mask = gt | (eq & (csum_eq <= quota))
    maskf = jnp.where(mask, 1.0, 0.0)
    pos = _dot(maskf.astype(_BF), tri) - 1.0

    sub_iota = lax.broadcasted_iota(jnp.int32, (PN, NPG), 0).astype(_F32)
    cf = jnp.where(mask & (pos == sub_iota), 1.0, 0.0)
    x0r = _mm_onehot(cf, xg)
    ones11 = jnp.ones((1, 1), _BF)
    dn = (((0,), (0,)), ((), ()))
    sh, sm_, sl = _split3(score)
    score_col = (lax.dot_general(sh, ones11, dn, preferred_element_type=_F32)
                 + lax.dot_general(sm_, ones11, dn, preferred_element_type=_F32)
                 + lax.dot_general(sl, ones11, dn, preferred_element_type=_F32))
    s0 = _mm_onehot(cf, score_col)
    x0 = x0r * s0

    eye_bf = (lax.broadcasted_iota(jnp.int32, (PN, PN), 0)
              == lax.broadcasted_iota(jnp.int32, (PN, PN), 1)).astype(_BF)
    lane_iota = lax.broadcasted_iota(jnp.int32, (PN, PN), 1).astype(_F32)
    rowi = lax.broadcasted_iota(jnp.int32, (PN, PN), 0)
    coli = lax.broadcasted_iota(jnp.int32, (PN, PN), 1)
    pad_big = jnp.where((rowi == coli) | (coli >= KP),
                        jnp.float32(1e10), jnp.float32(0.0))

    a = _conv(x0, _d2_of(x0, eye_bf, pad_big), lane_iota,
              c1a_ref[...], c1b_ref[...], c1w2_ref[...], c1b1_ref, c1b2_ref)
    bb = _conv(a, _d2_of(a, eye_bf, pad_big), lane_iota,
               c2a_ref[...], c2b_ref[...], c2w2_ref[...], c2b1_ref, c2b2_ref)
    cc = _conv(bb, _d2_of(bb, eye_bf, pad_big), lane_iota,
               c3a_ref[...], c3b_ref[...], c3w2_ref[...], c3b1_ref, c3b2_ref)
    dd = _conv(cc, _d2_of(cc, eye_bf, pad_big), lane_iota,
               c4a_ref[...], c4b_ref[...], c4w2_ref[...], c4b1_ref, c4b2_ref)

    h1 = _lrelu(_mm(x0, n1x_ref[...]) + _mm(a, n1a_ref[...])
                + _mm(bb, n1b_ref[...]) + _mm(cc, n1c_ref[...])
                + _mm(dd, n1d_ref[...]) + n1bias_ref[...])
    h2 = _mm(h1, n2_ref[...]) + n2b_ref[...]
    live = lax.broadcasted_iota(jnp.int32, (PN, 1), 0) < KP
    mx = jnp.max(jnp.where(live, h2, jnp.float32(-1e30)), axis=0, keepdims=True)
    mn = jnp.min(jnp.where(live, h2, jnp.float32(1e30)), axis=0, keepdims=True)
    sm = jnp.sum(jnp.where(live, h2, jnp.float32(0.0)), axis=0, keepdims=True)
    mean = sm / jnp.float32(KP)
    h3 = _lrelu(_mm(_lrelu(mx), n3a_ref[...]) + _mm(_lrelu(mn), n3b_ref[...])
                + _mm(_lrelu(sm), n3c_ref[...]) + _mm(_lrelu(mean), n3d_ref[...])
                + n3bias_ref[...])
    res = jnp.sum(h3 * n4_ref[...], axis=1, keepdims=True) + n4b_ref[...]
    out_ref[0] = jnp.broadcast_to(res, (1, 128))


def _run_graphs(xg3, xgt3, agg3, wrel, wroot, relb, wl):
    full = lambda shape: pl.BlockSpec(shape, lambda g: tuple(0 for _ in shape))
    in_specs = [
        pl.BlockSpec((1, NPG, 4), lambda g: (g, 0, 0)),
        pl.BlockSpec((1, 4, NPG), lambda g: (g, 0, 0)),
        pl.BlockSpec((1, 8, NPG), lambda g: (g, 0, 0)),
        pl.BlockSpec(memory_space=pltpu.SMEM),
        pl.BlockSpec(memory_space=pltpu.SMEM),
        pl.BlockSpec(memory_space=pltpu.SMEM),
    ] + [full(w.shape) for w in wl]
    out = pl.pallas_call(
        _graph_body,
        grid=(B,),
        out_shape=jax.ShapeDtypeStruct((B, 1, 128), _F32),
        in_specs=in_specs,
        out_specs=pl.BlockSpec((1, 1, 128), lambda g: (g, 0, 0)),
    )(xg3, xgt3, agg3, wrel, wroot, relb, *wl)
    return out[:, 0, 0:1]


E_PAD = 1605632
EPG = 200704
NCH = 196
ZS = NB // 16


def _sc_body(xt_hbm, src_hbm, dst_hbm, out_hbm,
             xcol_v, src_v, dst_v, vals_v, acc):
    c = lax.axis_index("c")
    s = lax.axis_index("s")
    d = s % 4
    g = c * 4 + s // 4
    zbase = s * ZS

    @pl.loop(0, 1024, step=16)
    def _zero(i):
        vals_v[pl.ds(i, 16)] = jnp.zeros((16,), _F32)

    for d4 in range(4):
        for k in range(3):
            pltpu.sync_copy(vals_v,
                            acc.at[pl.ds(d4 * NB + zbase + k * 1024, 1024)])
        pltpu.sync_copy(vals_v.at[pl.ds(0, 64)],
                        acc.at[pl.ds(d4 * NB + zbase + 3072, 64)])
    pltpu.sync_copy(xt_hbm.at[pl.ds(d * NB, NB)], xcol_v)
    plsc.subcore_barrier()

    ebase = g * EPG
    doff = d * NB

    @pl.loop(0, NCH)
    def _chunk(ch):
        off = ebase + ch * 1024
        pltpu.sync_copy(src_hbm.at[pl.ds(off, 1024)], src_v)
        row = g * (EPG // 128) + ch * 8
        pltpu.sync_copy(dst_hbm.at[pl.ds(row, 8)], dst_v)

        @pl.loop(0, 1024, step=16)
        def _gather(i):
            idx = src_v[pl.ds(i, 16)]
            vals_v[pl.ds(i, 16)] = plsc.load_gather(xcol_v, [idx])

        @pl.loop(0, 8)
        def _shift(j):
            @pl.loop(0, 128, step=16)
            def _sh16(i):
                dst_v[j, pl.ds(i, 16)] = dst_v[j, pl.ds(i, 16)] + doff

        for j in range(8):
            pltpu.sync_copy(vals_v.at[pl.ds(j * 128, 128)],
                            acc.at[dst_v.at[j]], add=True)

    plsc.subcore_barrier()
    obase = c * 4 * NB
    for d4 in range(4):
        for k in range(3):
            sl = pl.ds(d4 * NB + zbase + k * 1024, 1024)
            osl = pl.ds(obase + d4 * NB + zbase + k * 1024, 1024)
            pltpu.sync_copy(acc.at[sl], vals_v)
            pltpu.sync_copy(vals_v, out_hbm.at[osl])
        sl = pl.ds(d4 * NB + zbase + 3072, 64)
        osl = pl.ds(obase + d4 * NB + zbase + 3072, 64)
        pltpu.sync_copy(acc.at[sl], vals_v.at[pl.ds(0, 64)])
        pltpu.sync_copy(vals_v.at[pl.ds(0, 64)], out_hbm.at[osl])


def _edge_aggregate(x, src, dst):
    npad = E_PAD - N_EDGES
    srcp = jnp.concatenate([src, jnp.full((npad,), N_NODES, jnp.int32)])
    dstp = jnp.concatenate([dst, jnp.full((npad,), N_NODES, jnp.int32)])
    dst2 = dstp.reshape(E_PAD // 128, 128)
    xt_pad = jnp.pad(x.T, ((0, 0), (0, NB - N_NODES))).reshape(4 * NB)

    mesh = plsc.VectorSubcoreMesh(core_axis_name="c", subcore_axis_name="s")
    cp = pltpu.CompilerParams()
    if "needs_layout_passes" in pltpu.CompilerParams.__dataclass_fields__:
        cp = dataclasses.replace(cp, needs_layout_passes=False)
    f = pl.kernel(
        _sc_body,
        out_type=jax.ShapeDtypeStruct((2 * 4 * NB,), _F32),
        mesh=mesh,
        compiler_params=cp,
        scratch_types=[
            pltpu.VMEM((NB,), _F32),
            pltpu.VMEM((1024,), jnp.int32),
            pltpu.VMEM((8, 128), jnp.int32),
            pltpu.VMEM((1024,), _F32),
            pltpu.VMEM_SHARED((4 * NB,), _F32),
        ],
    )
    return f(xt_pad, srcp, dst2).reshape(2, 4, NB)


def kernel(x, params, edge_index, batch):
    p = params
    aggP = _edge_aggregate(x, edge_index[0], edge_index[1])
    agg3 = (aggP[:, :, :N_NODES].reshape(2, 4, B, NPG)
            .transpose(2, 0, 1, 3).reshape(B, 8, NPG))

    xg3 = x.reshape(B, NPG, 4)
    xgt3 = xg3.swapaxes(1, 2)

    def c_w(i):
        w1 = p['c%d_w1' % i]
        d = w1.shape[1] // 2
        w1a, w1b = w1[:, :d], w1[:, d:]
        return [w1a.T, w1b.T, p['c%d_w2' % i].T,
                p['c%d_b1' % i][None, :], p['c%d_b2' % i][None, :]]

    n1 = p['nn1_w'].T
    n1parts = [n1[0:4], n1[4:196], n1[196:388], n1[388:580], n1[580:772]]
    n3 = p['nn3_w'].T
    n3parts = [n3[0:192], n3[192:384], n3[384:576], n3[576:768]]

    wl = (c_w(1) + c_w(2) + c_w(3) + c_w(4)
          + n1parts + [p['nn1_b'][None, :], p['nn2_w'].T, p['nn2_b'][None, :]]
          + n3parts + [p['nn3_b'][None, :], p['nn4_w'], p['nn4_b'][None, :]])
    wl = [w.astype(_F32) for w in wl]

    return _run_graphs(xg3, xgt3, agg3,
                       p['sag_rel_w'], p['sag_root_w'],
                       p['sag_rel_b'][None, :], wl)
